# Initial kernel scaffold; baseline (speedup 1.0000x reference)
import jax
import jax.numpy as jnp
from jax.experimental import pallas as pl


def kernel(x, edge_index, edge_attr, batch, W_rel1, b_rel1, W_root1, W_rel2, b_rel2, W_root2, W1, b1, W2, b2, W3, b3):
    raise NotImplementedError("write your pallas kernel here")


# speedup vs baseline: 1.5577x; 1.5577x over previous
import functools

import jax
import jax.numpy as jnp
from jax import lax
from jax.experimental import pallas as pl
from jax.experimental.pallas import tpu as pltpu
from jax.experimental.pallas import tpu_sc as plsc

N = 10000
E = 320000
F = 128
H = 512
G = 64
NC = 2
NS = 16
NW = NC * NS
B = 128
NB = 80
EPW = B * NB
E_PAD = NW * EPW
WT = 10
RPT = N // WT
ZB = 200
BN = 2000


def _sc_agg(table, src, dst, w, C):
    mesh = plsc.VectorSubcoreMesh(core_axis_name="c", subcore_axis_name="s")

    @functools.partial(
        pl.kernel,
        out_type=jax.ShapeDtypeStruct((C * NC * N, F), jnp.float32),
        mesh=mesh,
        scratch_types=[
            pltpu.VMEM((B,), jnp.int32),
            pltpu.VMEM((B,), jnp.int32),
            pltpu.VMEM((B,), jnp.float32),
            pltpu.VMEM((B, F), jnp.float32),
            pltpu.VMEM((ZB, F), jnp.float32),
            pltpu.VMEM_SHARED((N, F), jnp.float32),
            pltpu.SemaphoreType.DMA,
        ],
    )
    def k(table_h, src_h, dst_h, w_h, out_h,
          src_v, dst_v, w_v, rows_v, zeros_v, acc_s, sem):
        cid = lax.axis_index("c")
        sid = lax.axis_index("s")
        wid = sid * NC + cid
        base = wid * EPW

        def zrow(j, carry):
            for f8 in range(F // 16):
                zeros_v[j, pl.ds(f8 * 16, 16)] = jnp.zeros((16,), jnp.float32)
            return carry
        lax.fori_loop(0, ZB, zrow, 0)

        for c in range(C):
            @pl.when(sid < WT)
            def _zero():
                for z in range(RPT // ZB):
                    pltpu.sync_copy(zeros_v,
                                    acc_s.at[pl.ds(sid * RPT + z * ZB, ZB)])
            plsc.subcore_barrier()

            def body(g, carry):
                off = base + g * B
                pltpu.sync_copy(src_h.at[pl.ds(off, B)], src_v)
                pltpu.sync_copy(dst_h.at[pl.ds(off, B)], dst_v)
                pltpu.sync_copy(w_h.at[pl.ds(off, B)], w_v)
                if c > 0:
                    for q in range(B // 16):
                        sl = pl.ds(q * 16, 16)
                        src_v[sl] = src_v[sl] + (c * N)
                pltpu.async_copy(table_h.at[src_v], rows_v, sem).wait()

                def scale(q, inner):
                    wvec = w_v[pl.ds(q * 16, 16)]
                    for j16 in range(16):
                        wj = wvec[j16]
                        j = q * 16 + j16
                        for f8 in range(F // 16):
                            sl = pl.ds(f8 * 16, 16)
                            rows_v[j, sl] = rows_v[j, sl] * wj
                    return inner
                lax.fori_loop(0, B // 16, scale, 0)

                pltpu.sync_copy(rows_v, acc_s.at[dst_v], add=True)
                return carry
            lax.fori_loop(0, NB, body, 0)
            plsc.subcore_barrier()

            @pl.when(sid < WT)
            def _writeout():
                out_row = (c * NC + cid) * N + sid * RPT
                pltpu.sync_copy(acc_s.at[pl.ds(sid * RPT, RPT)],
                                out_h.at[pl.ds(out_row, RPT)])
            plsc.subcore_barrier()

    return k(table, src, dst, w)


def _tc_layer1(acc, x, W_rel, W_root, b_rel):
    def body(acc_ref, x_ref, wr_ref, wt_ref, b_ref, out_ref):
        agg = acc_ref[0] + acc_ref[1]
        pre = lax.dot_general(agg, wr_ref[...], (((1,), (1,)), ((), ())),
                              preferred_element_type=jnp.float32, precision=lax.Precision.HIGHEST)
        pre = pre + lax.dot_general(x_ref[...], wt_ref[...],
                                    (((1,), (1,)), ((), ())),
                                    preferred_element_type=jnp.float32, precision=lax.Precision.HIGHEST)
        out_ref[0] = jnp.maximum(pre + b_ref[0], 0.0)

    return pl.pallas_call(
        body,
        grid=(4, N // BN),
        in_specs=[
            pl.BlockSpec((2, BN, F), lambda c, i: (0, i, 0)),
            pl.BlockSpec((BN, F), lambda c, i: (i, 0)),
            pl.BlockSpec((F, F), lambda c, i: (c, 0)),
            pl.BlockSpec((F, F), lambda c, i: (c, 0)),
            pl.BlockSpec((1, 1, F), lambda c, i: (c, 0, 0)),
        ],
        out_specs=pl.BlockSpec((1, BN, F), lambda c, i: (c, i, 0)),
        out_shape=jax.ShapeDtypeStruct((4, N, F), jnp.float32),
    )(acc, x, W_rel, W_root, b_rel.reshape(4, 1, F))


def _tc_layer2_pool(acc2, h1b, W_rel, W_root, b_rel, batch2d):
    def body(acc_ref, h1_ref, wr_ref, wt_ref, b_ref, bat_ref,
             pooled_ref, counts_ref):
        i = pl.program_id(0)
        wr = wr_ref[...]
        wt = wt_ref[...]
        total = jnp.zeros((BN, H), jnp.float32)
        for c in range(4):
            aggc = acc_ref[c, 0] + acc_ref[c, 1]
            total = total + lax.dot_general(
                aggc, wr[:, c * F:(c + 1) * F], (((1,), (1,)), ((), ())),
                preferred_element_type=jnp.float32, precision=lax.Precision.HIGHEST)
            total = total + lax.dot_general(
                h1_ref[c], wt[:, c * F:(c + 1) * F], (((1,), (1,)), ((), ())),
                preferred_element_type=jnp.float32, precision=lax.Precision.HIGHEST)
        h2 = jnp.maximum(total + b_ref[...], 0.0)
        bat = bat_ref[0, 0]
        gids = lax.broadcasted_iota(jnp.int32, (G, BN), 0)
        mask = (bat[None, :] == gids).astype(jnp.float32)
        psum = lax.dot_general(mask, h2, (((1,), (0,)), ((), ())),
                               preferred_element_type=jnp.float32, precision=lax.Precision.HIGHEST)
        cnt = lax.dot_general(mask, jnp.ones((BN, H), jnp.float32),
                              (((1,), (0,)), ((), ())),
                              preferred_element_type=jnp.float32, precision=lax.Precision.HIGHEST)

        @pl.when(i == 0)
        def _():
            pooled_ref[...] = jnp.zeros_like(pooled_ref)
            counts_ref[...] = jnp.zeros_like(counts_ref)

        pooled_ref[...] += psum
        counts_ref[...] += cnt

    return pl.pallas_call(
        body,
        grid=(N // BN,),
        in_specs=[
            pl.BlockSpec((4, 2, BN, F), lambda i: (0, 0, i, 0)),
            pl.BlockSpec((4, BN, F), lambda i: (0, i, 0)),
            pl.BlockSpec((H, H), lambda i: (0, 0)),
            pl.BlockSpec((H, H), lambda i: (0, 0)),
            pl.BlockSpec((1, H), lambda i: (0, 0)),
            pl.BlockSpec((1, 1, BN), lambda i: (i, 0, 0)),
        ],
        out_specs=[
            pl.BlockSpec((G, H), lambda i: (0, 0)),
            pl.BlockSpec((G, H), lambda i: (0, 0)),
        ],
        out_shape=[
            jax.ShapeDtypeStruct((G, H), jnp.float32),
            jax.ShapeDtypeStruct((G, H), jnp.float32),
        ],
    )(acc2, h1b, W_rel, W_root, b_rel.reshape(1, H), batch2d)


def _tc_head(pooled, counts, W1, b1, W2, b2, W3, b3):
    def body(p_ref, c_ref, w1_ref, b1_ref, w2_ref, b2_ref, w3_ref, b3_ref,
             out_ref):
        pm = p_ref[...] / jnp.maximum(c_ref[...], 1.0)
        z = lax.dot_general(pm, w1_ref[...], (((1,), (1,)), ((), ())),
                            preferred_element_type=jnp.float32, precision=lax.Precision.HIGHEST) + b1_ref[...]
        z = jnp.maximum(z, 0.0)
        z = lax.dot_general(z, w2_ref[...], (((1,), (1,)), ((), ())),
                            preferred_element_type=jnp.float32, precision=lax.Precision.HIGHEST) + b2_ref[...]
        z = jnp.maximum(z, 0.0)
        out_ref[...] = (jnp.sum(z * w3_ref[...], axis=1, keepdims=True)
                        + b3_ref[...])

    W2p = jnp.zeros((128, G), jnp.float32).at[:16].set(W2)
    b2p = jnp.zeros((1, 128), jnp.float32).at[:, :16].set(b2.reshape(1, 16))
    W3p = jnp.zeros((1, 128), jnp.float32).at[:, :16].set(W3)
    return pl.pallas_call(
        body,
        out_shape=jax.ShapeDtypeStruct((G, 1), jnp.float32),
    )(pooled, counts, W1, b1.reshape(1, G), W2p, b2p, W3p, b3.reshape(1, 1))


def kernel(x, edge_index, edge_attr, batch,
           W_rel1, b_rel1, W_root1,
           W_rel2, b_rel2, W_root2,
           W1, b1, W2, b2, W3, b3):
    pad = E_PAD - E
    src_p = jnp.concatenate([edge_index[0], jnp.zeros((pad,), jnp.int32)])
    dst_p = jnp.concatenate([edge_index[1], jnp.zeros((pad,), jnp.int32)])
    w_p = jnp.concatenate([edge_attr, jnp.zeros((pad,), jnp.float32)])

    acc1 = _sc_agg(x, src_p, dst_p, w_p, C=1).reshape(2, N, F)
    h1b = _tc_layer1(acc1, x, W_rel1, W_root1, b_rel1)
    acc2 = _sc_agg(h1b.reshape(4 * N, F), src_p, dst_p, w_p,
                   C=4).reshape(4, 2, N, F)
    pooled, counts = _tc_layer2_pool(acc2, h1b, W_rel2, W_root2, b_rel2,
                                     batch.reshape(N // BN, 1, BN))
    return _tc_head(pooled, counts, W1, b1, W2, b2, W3, b3)


# speedup vs baseline: 1.8815x; 1.2079x over previous
import functools

import jax
import jax.numpy as jnp
from jax import lax
from jax.experimental import pallas as pl
from jax.experimental.pallas import tpu as pltpu
from jax.experimental.pallas import tpu_sc as plsc

N = 10000
E = 320000
F = 128
H = 512
G = 64
NC = 2
NS = 16
NW = NC * NS
B = 128
NB = 80
EPW = B * NB
E_PAD = NW * EPW
WT = 10
RPT = N // WT
ZB = 40
BN = 2000


D = 2


def _sc_agg(table, idx_pack, w_pack, C):
    mesh = plsc.VectorSubcoreMesh(core_axis_name="c", subcore_axis_name="s")

    @functools.partial(
        pl.kernel,
        out_type=jax.ShapeDtypeStruct((C * NC * N, F), jnp.float32),
        mesh=mesh,
        scratch_types=(
            [pltpu.VMEM((2, B), jnp.int32) for _ in range(D)]
            + [pltpu.VMEM((B,), jnp.float32) for _ in range(D)]
            + [pltpu.VMEM((B, F), jnp.float32) for _ in range(D)]
            + [pltpu.VMEM((ZB, F), jnp.float32)]
            + [pltpu.VMEM_SHARED((N, F), jnp.float32)]
            + [pltpu.SemaphoreType.DMA for _ in range(2 * D)]
        ),
    )
    def k(table_h, idx_h, w_h, out_h,
          i0, i1, w0, w1, r0, r1,
          zeros_v, acc_s,
          g0, g1, s0, s1):
        idxs = [i0, i1]
        ws = [w0, w1]
        rows = [r0, r1]
        semg = [g0, g1]
        sems = [s0, s1]
        cid = lax.axis_index("c")
        sid = lax.axis_index("s")
        wid = sid * NC + cid
        kbase = wid * NB

        def zrow(j, carry):
            for f8 in range(F // 16):
                zeros_v[j, pl.ds(f8 * 16, 16)] = jnp.zeros((16,), jnp.float32)
            return carry
        lax.fori_loop(0, ZB, zrow, 0)

        def fetch(g, b, coff):
            pltpu.sync_copy(idx_h.at[kbase + g], idxs[b])
            pltpu.sync_copy(w_h.at[kbase + g], ws[b])
            for q in range(B // 16):
                sl = pl.ds(q * 16, 16)
                idxs[b][0, sl] = idxs[b][0, sl] + coff
            pltpu.async_copy(table_h.at[idxs[b].at[0]], rows[b], semg[b])

        def chunk_body(c, carry):
            coff = c * N
            @pl.when(sid < WT)
            def _zero():
                for z in range(RPT // ZB):
                    pltpu.sync_copy(zeros_v,
                                    acc_s.at[pl.ds(sid * RPT + z * ZB, ZB)])
            plsc.subcore_barrier()

            fetch(0, 0, coff)
            pltpu.async_copy(table_h.at[idxs[0].at[0]], rows[1], sems[1])

            def step(t, carry):
                for b in range(D):
                    g = t * D + b
                    bb = 1 - b
                    pltpu.make_async_copy(table_h.at[idxs[b].at[0]],
                                          rows[b], semg[b]).wait()
                    pltpu.make_async_copy(rows[bb],
                                          acc_s.at[idxs[bb].at[1]],
                                          sems[bb]).wait()
                    fetch((g + 1) % NB, bb, coff)

                    def scale(q, inner):
                        wvec = ws[b][pl.ds(q * 16, 16)]
                        for j16 in range(16):
                            wj = wvec[j16]
                            j = q * 16 + j16
                            for f8 in range(F // 16):
                                sl = pl.ds(f8 * 16, 16)
                                rows[b][j, sl] = rows[b][j, sl] * wj
                        return inner
                    lax.fori_loop(0, B // 16, scale, 0)

                    pltpu.async_copy(rows[b], acc_s.at[idxs[b].at[1]],
                                     sems[b], add=True)
                return carry
            lax.fori_loop(0, NB // D, step, 0)

            pltpu.make_async_copy(table_h.at[idxs[0].at[0]],
                                  rows[0], semg[0]).wait()
            pltpu.make_async_copy(rows[1], acc_s.at[idxs[1].at[1]],
                                  sems[1]).wait()
            plsc.subcore_barrier()

            @pl.when(sid < WT)
            def _writeout():
                out_row = (c * NC + cid) * N + sid * RPT
                pltpu.sync_copy(acc_s.at[pl.ds(sid * RPT, RPT)],
                                out_h.at[pl.ds(out_row, RPT)])
            plsc.subcore_barrier()
            return carry
        lax.fori_loop(0, C, chunk_body, 0)

    return k(table, idx_pack, w_pack)


def _tc_layer1(acc, x, W_rel, W_root, b_rel):
    def body(acc_ref, x_ref, wr_ref, wt_ref, b_ref, out_ref):
        agg = acc_ref[0] + acc_ref[1]
        pre = lax.dot_general(agg, wr_ref[...], (((1,), (1,)), ((), ())),
                              preferred_element_type=jnp.float32, precision=lax.Precision.HIGHEST)
        pre = pre + lax.dot_general(x_ref[...], wt_ref[...],
                                    (((1,), (1,)), ((), ())),
                                    preferred_element_type=jnp.float32, precision=lax.Precision.HIGHEST)
        out_ref[0] = jnp.maximum(pre + b_ref[0], 0.0)

    return pl.pallas_call(
        body,
        grid=(4, N // BN),
        in_specs=[
            pl.BlockSpec((2, BN, F), lambda c, i: (0, i, 0)),
            pl.BlockSpec((BN, F), lambda c, i: (i, 0)),
            pl.BlockSpec((F, F), lambda c, i: (c, 0)),
            pl.BlockSpec((F, F), lambda c, i: (c, 0)),
            pl.BlockSpec((1, 1, F), lambda c, i: (c, 0, 0)),
        ],
        out_specs=pl.BlockSpec((1, BN, F), lambda c, i: (c, i, 0)),
        out_shape=jax.ShapeDtypeStruct((4, N, F), jnp.float32),
    )(acc, x, W_rel, W_root, b_rel.reshape(4, 1, F))


def _tc_layer2_pool(acc2, h1b, W_rel, W_root, b_rel, batch2d):
    def body(acc_ref, h1_ref, wr_ref, wt_ref, b_ref, bat_ref,
             pooled_ref, counts_ref):
        i = pl.program_id(0)
        wr = wr_ref[...]
        wt = wt_ref[...]
        total = jnp.zeros((BN, H), jnp.float32)
        for c in range(4):
            aggc = acc_ref[c, 0] + acc_ref[c, 1]
            total = total + lax.dot_general(
                aggc, wr[:, c * F:(c + 1) * F], (((1,), (1,)), ((), ())),
                preferred_element_type=jnp.float32, precision=lax.Precision.HIGHEST)
            total = total + lax.dot_general(
                h1_ref[c], wt[:, c * F:(c + 1) * F], (((1,), (1,)), ((), ())),
                preferred_element_type=jnp.float32, precision=lax.Precision.HIGHEST)
        h2 = jnp.maximum(total + b_ref[...], 0.0)
        bat = bat_ref[0, 0]
        gids = lax.broadcasted_iota(jnp.int32, (G, BN), 0)
        mask = (bat[None, :] == gids).astype(jnp.float32)
        psum = lax.dot_general(mask, h2, (((1,), (0,)), ((), ())),
                               preferred_element_type=jnp.float32, precision=lax.Precision.HIGHEST)
        cnt = lax.dot_general(mask, jnp.ones((BN, H), jnp.float32),
                              (((1,), (0,)), ((), ())),
                              preferred_element_type=jnp.float32, precision=lax.Precision.HIGHEST)

        @pl.when(i == 0)
        def _():
            pooled_ref[...] = jnp.zeros_like(pooled_ref)
            counts_ref[...] = jnp.zeros_like(counts_ref)

        pooled_ref[...] += psum
        counts_ref[...] += cnt

    return pl.pallas_call(
        body,
        grid=(N // BN,),
        in_specs=[
            pl.BlockSpec((4, 2, BN, F), lambda i: (0, 0, i, 0)),
            pl.BlockSpec((4, BN, F), lambda i: (0, i, 0)),
            pl.BlockSpec((H, H), lambda i: (0, 0)),
            pl.BlockSpec((H, H), lambda i: (0, 0)),
            pl.BlockSpec((1, H), lambda i: (0, 0)),
            pl.BlockSpec((1, 1, BN), lambda i: (i, 0, 0)),
        ],
        out_specs=[
            pl.BlockSpec((G, H), lambda i: (0, 0)),
            pl.BlockSpec((G, H), lambda i: (0, 0)),
        ],
        out_shape=[
            jax.ShapeDtypeStruct((G, H), jnp.float32),
            jax.ShapeDtypeStruct((G, H), jnp.float32),
        ],
    )(acc2, h1b, W_rel, W_root, b_rel.reshape(1, H), batch2d)


def _tc_head(pooled, counts, W1, b1, W2, b2, W3, b3):
    def body(p_ref, c_ref, w1_ref, b1_ref, w2_ref, b2_ref, w3_ref, b3_ref,
             out_ref):
        pm = p_ref[...] / jnp.maximum(c_ref[...], 1.0)
        z = lax.dot_general(pm, w1_ref[...], (((1,), (1,)), ((), ())),
                            preferred_element_type=jnp.float32, precision=lax.Precision.HIGHEST) + b1_ref[...]
        z = jnp.maximum(z, 0.0)
        z = lax.dot_general(z, w2_ref[...], (((1,), (1,)), ((), ())),
                            preferred_element_type=jnp.float32, precision=lax.Precision.HIGHEST) + b2_ref[...]
        z = jnp.maximum(z, 0.0)
        out_ref[...] = (jnp.sum(z * w3_ref[...], axis=1, keepdims=True)
                        + b3_ref[...])

    W2p = jnp.zeros((128, G), jnp.float32).at[:16].set(W2)
    b2p = jnp.zeros((1, 128), jnp.float32).at[:, :16].set(b2.reshape(1, 16))
    W3p = jnp.zeros((1, 128), jnp.float32).at[:, :16].set(W3)
    return pl.pallas_call(
        body,
        out_shape=jax.ShapeDtypeStruct((G, 1), jnp.float32),
    )(pooled, counts, W1, b1.reshape(1, G), W2p, b2p, W3p, b3.reshape(1, 1))


def kernel(x, edge_index, edge_attr, batch,
           W_rel1, b_rel1, W_root1,
           W_rel2, b_rel2, W_root2,
           W1, b1, W2, b2, W3, b3):
    pad = E_PAD - E
    src_p = jnp.concatenate([edge_index[0], jnp.zeros((pad,), jnp.int32)])
    dst_p = jnp.concatenate([edge_index[1], jnp.zeros((pad,), jnp.int32)])
    w_p = jnp.concatenate([edge_attr, jnp.zeros((pad,), jnp.float32)])
    idx_pack = jnp.stack([src_p.reshape(NW * NB, B),
                          dst_p.reshape(NW * NB, B)], axis=1)
    w_pack = w_p.reshape(NW * NB, B)

    acc1 = _sc_agg(x, idx_pack, w_pack, C=1).reshape(2, N, F)
    h1b = _tc_layer1(acc1, x, W_rel1, W_root1, b_rel1)
    acc2 = _sc_agg(h1b.reshape(4 * N, F), idx_pack, w_pack,
                   C=4).reshape(4, 2, N, F)
    pooled, counts = _tc_layer2_pool(acc2, h1b, W_rel2, W_root2, b_rel2,
                                     batch.reshape(N // BN, 1, BN))
    return _tc_head(pooled, counts, W1, b1, W2, b2, W3, b3)
